# baseline (speedup 1.0000x reference)
import functools

import jax
import jax.numpy as jnp
from jax import lax
from jax.experimental import pallas as pl
from jax.experimental.pallas import tpu as pltpu
from jax.experimental.pallas import tpu_sc as plsc

F32 = jnp.float32
NC = 2
NS = 16
NW = NC * NS
C = 80


def _tc_qkv(x, wq_w, wq_b, wk_w, wv_w):
    n, d = x.shape
    bn = 2000

    def body(x_ref, wq_ref, wqb_ref, wk_ref, wv_ref, xq_ref, xk_ref, xv_ref):
        xb = x_ref[...]
        xq_ref[...] = jnp.dot(xb, wq_ref[...], preferred_element_type=F32) + wqb_ref[...]
        xk_ref[...] = jnp.dot(xb, wk_ref[...], preferred_element_type=F32)
        xv_ref[...] = jnp.dot(xb, wv_ref[...], preferred_element_type=F32)

    out = jax.ShapeDtypeStruct((n, d), F32)
    w_spec = pl.BlockSpec((d, d), lambda i: (0, 0))
    b_spec = pl.BlockSpec((1, d), lambda i: (0, 0))
    r_spec = pl.BlockSpec((bn, d), lambda i: (i, 0))
    return pl.pallas_call(
        body,
        grid=(n // bn,),
        in_specs=[r_spec, w_spec, b_spec, w_spec, w_spec],
        out_specs=[r_spec, r_spec, r_spec],
        out_shape=[out, out, out],
    )(x, wq_w, wq_b.reshape(1, d), wk_w, wv_w)


def _edge_s(eab, g1b, wc1_ref, bc1_ref):
    d = eab.shape[1]
    big1 = jnp.dot(eab.astype(jnp.bfloat16), wc1_ref[...],
                   preferred_element_type=F32) + bc1_ref[...]
    t = g1b * big1[:, :d]
    s = jnp.sqrt(jnp.maximum(t, 0.0)) - jnp.sqrt(jnp.maximum(-t, 0.0))
    return jnp.maximum(s + big1[:, d:], 0.0)


def _tc_edge_p(ea, g1, wc1, bc1, m16b):
    e, d = ea.shape
    be = 2000

    def body(ea_ref, g1_ref, wc1_ref, bc1_ref, m_ref, p_ref):
        s = _edge_s(ea_ref[...], g1_ref[...], wc1_ref, bc1_ref)
        pp = jnp.dot(s.astype(jnp.bfloat16), m_ref[...],
                     preferred_element_type=F32)
        p_ref[...] = jnp.exp(jnp.clip(pp, -5.0, 5.0))

    r_spec = pl.BlockSpec((be, d), lambda i: (i, 0))
    return pl.pallas_call(
        body,
        grid=(e // be,),
        in_specs=[r_spec, r_spec,
                  pl.BlockSpec((d, 2 * d), lambda i: (0, 0)),
                  pl.BlockSpec((1, 2 * d), lambda i: (0, 0)),
                  pl.BlockSpec((d, 16), lambda i: (0, 0))],
        out_specs=pl.BlockSpec((be, 16), lambda i: (i, 0)),
        out_shape=jax.ShapeDtypeStruct((e, 16), F32),
    )(ea, g1, wc1, bc1, m16b)


def _tc_edge_out(ea, g1, wc1, bc1, weob, ln_eg, ln_eb):
    e, d = ea.shape
    be = 2000

    def body(ea_ref, g1_ref, wc1_ref, bc1_ref, weo_ref, g_ref, b_ref,
             eout_ref):
        eab = ea_ref[...]
        s = _edge_s(eab, g1_ref[...], wc1_ref, bc1_ref)
        eo = jnp.dot(s.astype(jnp.bfloat16), weo_ref[...],
                     preferred_element_type=F32) + eab
        mu = jnp.mean(eo, axis=-1, keepdims=True)
        var = jnp.mean((eo - mu) ** 2, axis=-1, keepdims=True)
        eout_ref[...] = (eo - mu) / jnp.sqrt(var + 1e-5) * g_ref[...] + b_ref[...]

    b_spec = pl.BlockSpec((1, d), lambda i: (0, 0))
    r_spec = pl.BlockSpec((be, d), lambda i: (i, 0))
    return pl.pallas_call(
        body,
        grid=(e // be,),
        in_specs=[r_spec, r_spec,
                  pl.BlockSpec((d, 2 * d), lambda i: (0, 0)),
                  pl.BlockSpec((1, 2 * d), lambda i: (0, 0)),
                  pl.BlockSpec((d, d), lambda i: (0, 0)),
                  b_spec, b_spec],
        out_specs=r_spec,
        out_shape=jax.ShapeDtypeStruct((e, d), F32),
    )(ea, g1, wc1, bc1, weob, ln_eg.reshape(1, d), ln_eb.reshape(1, d))


def _tc_node(u, den128, x, wo_w, ln_ng, ln_nb):
    n, d = x.shape

    def body(u_ref, den_ref, x_ref, wo_ref, g_ref, b_ref, h_ref):
        xo = u_ref[...] / (den_ref[...] + 1e-16)
        hh = jnp.dot(xo, wo_ref[...], preferred_element_type=F32) + x_ref[...]
        mu = jnp.mean(hh, axis=-1, keepdims=True)
        var = jnp.mean((hh - mu) ** 2, axis=-1, keepdims=True)
        h_ref[...] = (hh - mu) / jnp.sqrt(var + 1e-5) * g_ref[...] + b_ref[...]

    bn = 2000
    w_spec = pl.BlockSpec((d, d), lambda i: (0, 0))
    b_spec = pl.BlockSpec((1, d), lambda i: (0, 0))
    r_spec = pl.BlockSpec((bn, d), lambda i: (i, 0))
    return pl.pallas_call(
        body,
        grid=(n // bn,),
        in_specs=[r_spec, r_spec, r_spec, w_spec, b_spec, b_spec],
        out_specs=r_spec,
        out_shape=jax.ShapeDtypeStruct((n, d), F32),
    )(u, den128, x, wo_w, ln_ng.reshape(1, d), ln_nb.reshape(1, d))


def _sc_gather_add(xk, xq, src, dst):
    n, d = xk.shape
    e = src.shape[0]
    ew_ = e // NW
    ch = ew_ // C
    npair = ch // 2
    mesh = plsc.VectorSubcoreMesh(core_axis_name="c", subcore_axis_name="s")

    buf = lambda: [pltpu.VMEM((C,), jnp.int32), pltpu.VMEM((C,), jnp.int32),
                   pltpu.VMEM((C, d), F32), pltpu.VMEM((C, d), F32),
                   pltpu.VMEM((C, d), F32)]

    @functools.partial(
        pl.kernel,
        out_type=jax.ShapeDtypeStruct((e, d), F32),
        mesh=mesh,
        scratch_types=buf() + buf() + [pltpu.SemaphoreType.DMA] * 6,
        compiler_params=pltpu.CompilerParams(use_tc_tiling_on_sc=False,
                                             needs_layout_passes=False),
    )
    def k(xk_hbm, xq_hbm, src_hbm, dst_hbm, g1_hbm,
          sidx0, didx0, ka0, qa0, ob0, sidx1, didx1, ka1, qa1, ob1,
          semi0, semi1, semg0, semg1, semw0, semw1):
        wid = lax.axis_index("c") * NS + lax.axis_index("s")
        base = wid * ew_

        def idx_issue(i, sidx, didx, semi):
            off = base + i * C
            pltpu.async_copy(src_hbm.at[pl.ds(off, C)], sidx, semi)
            pltpu.async_copy(dst_hbm.at[pl.ds(off, C)], didx, semi)

        def idx_wait(i, sidx, didx, semi):
            off = base + i * C
            pltpu.make_async_copy(src_hbm.at[pl.ds(off, C)], sidx, semi).wait()
            pltpu.make_async_copy(dst_hbm.at[pl.ds(off, C)], didx, semi).wait()

        def gat_issue(sidx, didx, ka, qa, semg):
            pltpu.async_copy(xk_hbm.at[sidx], ka, semg)
            pltpu.async_copy(xq_hbm.at[didx], qa, semg)

        def gat_wait(sidx, didx, ka, qa, semg):
            pltpu.make_async_copy(xk_hbm.at[sidx], ka, semg).wait()
            pltpu.make_async_copy(xq_hbm.at[didx], qa, semg).wait()

        def add(ka, qa, ob):
            @plsc.parallel_loop(0, C, unroll=8)
            def row(r):
                for j in range(d // 16):
                    sl = pl.ds(j * 16, 16)
                    ob[r, sl] = ka[r, sl] + qa[r, sl]

        def w_issue(i, ob, semw):
            pltpu.async_copy(ob, g1_hbm.at[pl.ds(base + i * C, C)], semw)

        def w_wait(i, ob, semw):
            pltpu.make_async_copy(ob, g1_hbm.at[pl.ds(base + i * C, C)],
                                  semw).wait()

        idx_issue(0, sidx0, didx0, semi0)
        idx_issue(1, sidx1, didx1, semi1)
        idx_wait(0, sidx0, didx0, semi0)
        gat_issue(sidx0, didx0, ka0, qa0, semg0)

        def body(j, _):
            a = 2 * j
            b = a + 1
            idx_wait(b, sidx1, didx1, semi1)
            gat_issue(sidx1, didx1, ka1, qa1, semg1)
            gat_wait(sidx0, didx0, ka0, qa0, semg0)

            @pl.when(j > 0)
            def _():
                w_wait(a - 2, ob0, semw0)

            add(ka0, qa0, ob0)
            w_issue(a, ob0, semw0)
            idx_issue(a + 2, sidx0, didx0, semi0)
            gat_wait(sidx1, didx1, ka1, qa1, semg1)

            @pl.when(j > 0)
            def _():
                w_wait(b - 2, ob1, semw1)

            add(ka1, qa1, ob1)
            w_issue(b, ob1, semw1)

            @pl.when(b + 2 < ch)
            def _():
                idx_issue(b + 2, sidx1, didx1, semi1)

            idx_wait(a + 2, sidx0, didx0, semi0)
            gat_issue(sidx0, didx0, ka0, qa0, semg0)
            return 0

        lax.fori_loop(0, npair, body, 0)
        gat_wait(sidx0, didx0, ka0, qa0, semg0)
        w_wait(ch - 3, ob0, semw0)
        add(ka0, qa0, ob0)
        w_issue(ch - 1, ob0, semw0)
        w_wait(ch - 1, ob0, semw0)
        w_wait(ch - 2, ob1, semw1)

    return k(xk, xq, src, dst)


def _sc_segment(xv, src, dst, p16):
    n, d = xv.shape
    e = src.shape[0]
    dm = d + 16
    cs = 40
    ew_ = e // NW
    ch = ew_ // cs
    nch = n // cs
    mesh = plsc.VectorSubcoreMesh(core_axis_name="c", subcore_axis_name="s")

    @functools.partial(
        pl.kernel,
        out_type=jax.ShapeDtypeStruct((NC, n, dm), F32),
        mesh=mesh,
        scratch_types=(
            [pltpu.VMEM((cs,), jnp.int32), pltpu.VMEM((cs,), jnp.int32),
             pltpu.VMEM((cs,), jnp.int32), pltpu.VMEM((cs, 16), F32),
             pltpu.VMEM((cs, d), F32), pltpu.VMEM((cs, dm), F32)] * 2
            + [pltpu.VMEM_SHARED((n, dm), F32)]
            + [pltpu.SemaphoreType.DMA] * 6),
        compiler_params=pltpu.CompilerParams(use_tc_tiling_on_sc=False,
                                             needs_layout_passes=False),
    )
    def k(xv_hbm, src_hbm, dst_hbm, p_hbm, out_hbm,
          sidx0, didx0, ds0, pvv0, xvv0, msg0,
          sidx1, didx1, ds1, pvv1, xvv1, msg1,
          acc, semi0, semi1, semg0, semg1, sems0, sems1):
        cid = lax.axis_index("c")
        sid = lax.axis_index("s")
        wid = cid * NS + sid
        base = wid * ew_
        my_nch = (nch - sid + NS - 1) // NS

        def zrow(r, _):
            for j in range(dm // 16):
                msg0[r, pl.ds(j * 16, 16)] = jnp.zeros((16,), F32)
            return 0

        lax.fori_loop(0, cs, zrow, 0)

        def zchunk(j, _):
            pltpu.sync_copy(msg0, acc.at[pl.ds((sid + j * NS) * cs, cs)])
            return 0

        lax.fori_loop(0, my_nch, zchunk, 0)
        plsc.subcore_barrier()

        def idx_issue(i, sidx, didx, pvv, semi):
            off = base + i * cs
            pltpu.async_copy(src_hbm.at[pl.ds(off, cs)], sidx, semi)
            pltpu.async_copy(dst_hbm.at[pl.ds(off, cs)], didx, semi)
            pltpu.async_copy(p_hbm.at[pl.ds(off, cs)], pvv, semi)

        def idx_wait(i, sidx, didx, pvv, semi):
            off = base + i * cs
            pltpu.make_async_copy(src_hbm.at[pl.ds(off, cs)], sidx, semi).wait()
            pltpu.make_async_copy(dst_hbm.at[pl.ds(off, cs)], didx, semi).wait()
            pltpu.make_async_copy(p_hbm.at[pl.ds(off, cs)], pvv, semi).wait()

        def compute(xvv, pvv, msg, didx, dsv):
            @plsc.parallel_loop(0, cs, unroll=8)
            def row(r):
                prow = pvv[r, :]
                for h in range(d // 16):
                    sl = pl.ds(h * 16, 16)
                    pe = prow.at[jnp.full((16,), h, jnp.int32)].get(
                        mode="promise_in_bounds")
                    msg[r, sl] = xvv[r, sl] * pe
                msg[r, pl.ds(d, 16)] = prow
            offs = list(range(0, cs - 15, 16))
            if cs % 16:
                offs.append(cs - 16)
            for q in offs:
                sl = pl.ds(q, 16)
                dsv[sl] = didx[sl]

        def scat_issue(msg, dsv, sems):
            pltpu.async_copy(msg, acc.at[dsv], sems, add=True)

        def scat_wait(msg, dsv, sems):
            pltpu.make_async_copy(msg, acc.at[dsv], sems).wait()

        idx_issue(0, sidx0, didx0, pvv0, semi0)
        idx_issue(1, sidx1, didx1, pvv1, semi1)
        idx_wait(0, sidx0, didx0, pvv0, semi0)
        pltpu.async_copy(xv_hbm.at[sidx0], xvv0, semg0)

        def body(j, _):
            a = 2 * j
            b = a + 1
            idx_wait(b, sidx1, didx1, pvv1, semi1)
            pltpu.async_copy(xv_hbm.at[sidx1], xvv1, semg1)
            pltpu.make_async_copy(xv_hbm.at[sidx0], xvv0, semg0).wait()

            @pl.when(j > 0)
            def _():
                scat_wait(msg0, ds0, sems0)

            compute(xvv0, pvv0, msg0, didx0, ds0)
            scat_issue(msg0, ds0, sems0)

            @pl.when(a + 2 < ch)
            def _():
                idx_issue(a + 2, sidx0, didx0, pvv0, semi0)

            pltpu.make_async_copy(xv_hbm.at[sidx1], xvv1, semg1).wait()

            @pl.when(j > 0)
            def _():
                scat_wait(msg1, ds1, sems1)

            compute(xvv1, pvv1, msg1, didx1, ds1)
            scat_issue(msg1, ds1, sems1)

            @pl.when(b + 2 < ch)
            def _():
                idx_issue(b + 2, sidx1, didx1, pvv1, semi1)

            @pl.when(a + 2 < ch)
            def _():
                idx_wait(a + 2, sidx0, didx0, pvv0, semi0)
                pltpu.async_copy(xv_hbm.at[sidx0], xvv0, semg0)

            return 0

        lax.fori_loop(0, ch // 2, body, 0)
        scat_wait(msg0, ds0, sems0)
        scat_wait(msg1, ds1, sems1)
        plsc.subcore_barrier()

        def wchunk(j, _):
            row0 = (sid + j * NS) * cs
            pltpu.sync_copy(acc.at[pl.ds(row0, cs)],
                            out_hbm.at[cid, pl.ds(row0, cs)])
            return 0

        lax.fori_loop(0, my_nch, wchunk, 0)

    return k(xv, src, dst, p16)


def kernel(x, edge_index, edge_attr, wq_w, wq_b, wk_w, wv_w, web_w, web_b,
           wew_w, wew_b, wo_w, weo_w, Aw, ln_ng, ln_nb, ln_eg, ln_eb):
    n, d = x.shape
    hd, h, _ = Aw.shape
    src = edge_index[0]
    dst = edge_index[1]

    m16 = jnp.zeros((d, 16), F32).at[
        jnp.arange(d), jnp.arange(d) // hd].set(Aw[:, :, 0].T.reshape(-1))

    wc1 = jnp.concatenate([wew_w, web_w], axis=1).astype(jnp.bfloat16)
    bc1 = jnp.concatenate([wew_b, web_b]).reshape(1, 2 * d)
    m16b = m16.astype(jnp.bfloat16)
    weob = weo_w.astype(jnp.bfloat16)

    xq, xk, xv = _tc_qkv(x, wq_w, wq_b, wk_w, wv_w)
    g1 = _sc_gather_add(xk, xq, src, dst)
    p16 = _tc_edge_p(edge_attr, g1, wc1, bc1, m16b)
    parts = _sc_segment(xv, src, dst, p16)
    e_out = _tc_edge_out(edge_attr, g1, wc1, bc1, weob, ln_eg, ln_eb)
    u144 = parts[0] + parts[1]
    u = u144[:, :d]
    den128 = jnp.repeat(u144[:, d:d + h], hd, axis=1)
    hh = _tc_node(u, den128, x, wo_w, ln_ng, ln_nb)
    return (hh, e_out)


# speedup vs baseline: 1.0107x; 1.0107x over previous
import functools

import jax
import jax.numpy as jnp
from jax import lax
from jax.experimental import pallas as pl
from jax.experimental.pallas import tpu as pltpu
from jax.experimental.pallas import tpu_sc as plsc

F32 = jnp.float32
NC = 2
NS = 16
NW = NC * NS
C = 80


def _tc_qkv(x, wq_w, wq_b, wk_w, wv_w):
    n, d = x.shape
    bn = 2000

    def body(x_ref, wq_ref, wqb_ref, wk_ref, wv_ref, xq_ref, xk_ref, xv_ref):
        xb = x_ref[...]
        xq_ref[...] = jnp.dot(xb, wq_ref[...], preferred_element_type=F32) + wqb_ref[...]
        xk_ref[...] = jnp.dot(xb, wk_ref[...], preferred_element_type=F32)
        xv_ref[...] = jnp.dot(xb, wv_ref[...], preferred_element_type=F32)

    out = jax.ShapeDtypeStruct((n, d), F32)
    w_spec = pl.BlockSpec((d, d), lambda i: (0, 0))
    b_spec = pl.BlockSpec((1, d), lambda i: (0, 0))
    r_spec = pl.BlockSpec((bn, d), lambda i: (i, 0))
    return pl.pallas_call(
        body,
        grid=(n // bn,),
        in_specs=[r_spec, w_spec, b_spec, w_spec, w_spec],
        out_specs=[r_spec, r_spec, r_spec],
        out_shape=[out, out, out],
    )(x, wq_w, wq_b.reshape(1, d), wk_w, wv_w)


def _edge_s(eab, g1b, wc1_ref, bc1_ref):
    d = eab.shape[1]
    big1 = jnp.dot(eab.astype(jnp.bfloat16), wc1_ref[...],
                   preferred_element_type=F32) + bc1_ref[...]
    t = g1b * big1[:, :d]
    s = jnp.sqrt(jnp.maximum(t, 0.0)) - jnp.sqrt(jnp.maximum(-t, 0.0))
    return jnp.maximum(s + big1[:, d:], 0.0)


def _tc_edge(ea, g1, wc1, bc1, wc2, ln_eg, ln_eb):
    e, d = ea.shape
    be = 2000

    def body(ea_ref, g1_ref, wc1_ref, bc1_ref, wc2_ref, g_ref, b_ref,
             eout_ref, p_ref):
        eab = ea_ref[...]
        s = _edge_s(eab, g1_ref[...], wc1_ref, bc1_ref)
        big2 = jnp.dot(s.astype(jnp.bfloat16), wc2_ref[...],
                       preferred_element_type=F32)
        p_ref[...] = jnp.exp(jnp.clip(big2[:, d:d + 16], -5.0, 5.0))
        eo = big2[:, :d] + eab
        mu = jnp.mean(eo, axis=-1, keepdims=True)
        var = jnp.mean((eo - mu) ** 2, axis=-1, keepdims=True)
        eout_ref[...] = (eo - mu) / jnp.sqrt(var + 1e-5) * g_ref[...] + b_ref[...]

    b_spec = pl.BlockSpec((1, d), lambda i: (0, 0))
    r_spec = pl.BlockSpec((be, d), lambda i: (i, 0))
    return pl.pallas_call(
        body,
        grid=(e // be,),
        in_specs=[r_spec, r_spec,
                  pl.BlockSpec((d, 2 * d), lambda i: (0, 0)),
                  pl.BlockSpec((1, 2 * d), lambda i: (0, 0)),
                  pl.BlockSpec((d, 2 * d), lambda i: (0, 0)),
                  b_spec, b_spec],
        out_specs=[r_spec, pl.BlockSpec((be, 16), lambda i: (i, 0))],
        out_shape=[jax.ShapeDtypeStruct((e, d), F32),
                   jax.ShapeDtypeStruct((e, 16), F32)],
    )(ea, g1, wc1, bc1, wc2, ln_eg.reshape(1, d), ln_eb.reshape(1, d))


def _tc_node(u, den128, x, wo_w, ln_ng, ln_nb):
    n, d = x.shape

    def body(u_ref, den_ref, x_ref, wo_ref, g_ref, b_ref, h_ref):
        xo = u_ref[...] / (den_ref[...] + 1e-16)
        hh = jnp.dot(xo, wo_ref[...], preferred_element_type=F32) + x_ref[...]
        mu = jnp.mean(hh, axis=-1, keepdims=True)
        var = jnp.mean((hh - mu) ** 2, axis=-1, keepdims=True)
        h_ref[...] = (hh - mu) / jnp.sqrt(var + 1e-5) * g_ref[...] + b_ref[...]

    bn = 2000
    w_spec = pl.BlockSpec((d, d), lambda i: (0, 0))
    b_spec = pl.BlockSpec((1, d), lambda i: (0, 0))
    r_spec = pl.BlockSpec((bn, d), lambda i: (i, 0))
    return pl.pallas_call(
        body,
        grid=(n // bn,),
        in_specs=[r_spec, r_spec, r_spec, w_spec, b_spec, b_spec],
        out_specs=r_spec,
        out_shape=jax.ShapeDtypeStruct((n, d), F32),
    )(u, den128, x, wo_w, ln_ng.reshape(1, d), ln_nb.reshape(1, d))


def _sc_gather_add(xk, xq, src, dst):
    n, d = xk.shape
    e = src.shape[0]
    ew_ = e // NW
    ch = ew_ // C
    npair = ch // 2
    mesh = plsc.VectorSubcoreMesh(core_axis_name="c", subcore_axis_name="s")

    buf = lambda: [pltpu.VMEM((C,), jnp.int32), pltpu.VMEM((C,), jnp.int32),
                   pltpu.VMEM((C, d), F32), pltpu.VMEM((C, d), F32),
                   pltpu.VMEM((C, d), F32)]

    @functools.partial(
        pl.kernel,
        out_type=jax.ShapeDtypeStruct((e, d), F32),
        mesh=mesh,
        scratch_types=buf() + buf() + [pltpu.SemaphoreType.DMA] * 6,
        compiler_params=pltpu.CompilerParams(use_tc_tiling_on_sc=False,
                                             needs_layout_passes=False),
    )
    def k(xk_hbm, xq_hbm, src_hbm, dst_hbm, g1_hbm,
          sidx0, didx0, ka0, qa0, ob0, sidx1, didx1, ka1, qa1, ob1,
          semi0, semi1, semg0, semg1, semw0, semw1):
        wid = lax.axis_index("c") * NS + lax.axis_index("s")
        base = wid * ew_

        def idx_issue(i, sidx, didx, semi):
            off = base + i * C
            pltpu.async_copy(src_hbm.at[pl.ds(off, C)], sidx, semi)
            pltpu.async_copy(dst_hbm.at[pl.ds(off, C)], didx, semi)

        def idx_wait(i, sidx, didx, semi):
            off = base + i * C
            pltpu.make_async_copy(src_hbm.at[pl.ds(off, C)], sidx, semi).wait()
            pltpu.make_async_copy(dst_hbm.at[pl.ds(off, C)], didx, semi).wait()

        def gat_issue(sidx, didx, ka, qa, semg):
            pltpu.async_copy(xk_hbm.at[sidx], ka, semg)
            pltpu.async_copy(xq_hbm.at[didx], qa, semg)

        def gat_wait(sidx, didx, ka, qa, semg):
            pltpu.make_async_copy(xk_hbm.at[sidx], ka, semg).wait()
            pltpu.make_async_copy(xq_hbm.at[didx], qa, semg).wait()

        def add(ka, qa, ob):
            @plsc.parallel_loop(0, C, unroll=8)
            def row(r):
                for j in range(d // 16):
                    sl = pl.ds(j * 16, 16)
                    ob[r, sl] = ka[r, sl] + qa[r, sl]

        def w_issue(i, ob, semw):
            pltpu.async_copy(ob, g1_hbm.at[pl.ds(base + i * C, C)], semw)

        def w_wait(i, ob, semw):
            pltpu.make_async_copy(ob, g1_hbm.at[pl.ds(base + i * C, C)],
                                  semw).wait()

        idx_issue(0, sidx0, didx0, semi0)
        idx_issue(1, sidx1, didx1, semi1)
        idx_wait(0, sidx0, didx0, semi0)
        gat_issue(sidx0, didx0, ka0, qa0, semg0)

        def body(j, _):
            a = 2 * j
            b = a + 1
            idx_wait(b, sidx1, didx1, semi1)
            gat_issue(sidx1, didx1, ka1, qa1, semg1)
            gat_wait(sidx0, didx0, ka0, qa0, semg0)

            @pl.when(j > 0)
            def _():
                w_wait(a - 2, ob0, semw0)

            add(ka0, qa0, ob0)
            w_issue(a, ob0, semw0)
            idx_issue(a + 2, sidx0, didx0, semi0)
            gat_wait(sidx1, didx1, ka1, qa1, semg1)

            @pl.when(j > 0)
            def _():
                w_wait(b - 2, ob1, semw1)

            add(ka1, qa1, ob1)
            w_issue(b, ob1, semw1)

            @pl.when(b + 2 < ch)
            def _():
                idx_issue(b + 2, sidx1, didx1, semi1)

            idx_wait(a + 2, sidx0, didx0, semi0)
            gat_issue(sidx0, didx0, ka0, qa0, semg0)
            return 0

        lax.fori_loop(0, npair, body, 0)
        gat_wait(sidx0, didx0, ka0, qa0, semg0)
        w_wait(ch - 3, ob0, semw0)
        add(ka0, qa0, ob0)
        w_issue(ch - 1, ob0, semw0)
        w_wait(ch - 1, ob0, semw0)
        w_wait(ch - 2, ob1, semw1)

    return k(xk, xq, src, dst)


def _sc_segment(xv, src, dst, p16):
    n, d = xv.shape
    e = src.shape[0]
    dm = d + 16
    cs = 40
    ew_ = e // NW
    ch = ew_ // cs
    nch = n // cs
    mesh = plsc.VectorSubcoreMesh(core_axis_name="c", subcore_axis_name="s")

    @functools.partial(
        pl.kernel,
        out_type=jax.ShapeDtypeStruct((NC, n, dm), F32),
        mesh=mesh,
        scratch_types=(
            [pltpu.VMEM((cs,), jnp.int32), pltpu.VMEM((cs,), jnp.int32),
             pltpu.VMEM((cs,), jnp.int32), pltpu.VMEM((cs, 16), F32),
             pltpu.VMEM((cs, d), F32), pltpu.VMEM((cs, dm), F32)] * 2
            + [pltpu.VMEM_SHARED((n, dm), F32)]
            + [pltpu.SemaphoreType.DMA] * 6),
        compiler_params=pltpu.CompilerParams(use_tc_tiling_on_sc=False,
                                             needs_layout_passes=False),
    )
    def k(xv_hbm, src_hbm, dst_hbm, p_hbm, out_hbm,
          sidx0, didx0, ds0, pvv0, xvv0, msg0,
          sidx1, didx1, ds1, pvv1, xvv1, msg1,
          acc, semi0, semi1, semg0, semg1, sems0, sems1):
        cid = lax.axis_index("c")
        sid = lax.axis_index("s")
        wid = cid * NS + sid
        base = wid * ew_
        my_nch = (nch - sid + NS - 1) // NS

        def zrow(r, _):
            for j in range(dm // 16):
                msg0[r, pl.ds(j * 16, 16)] = jnp.zeros((16,), F32)
            return 0

        lax.fori_loop(0, cs, zrow, 0)

        def zchunk(j, _):
            pltpu.sync_copy(msg0, acc.at[pl.ds((sid + j * NS) * cs, cs)])
            return 0

        lax.fori_loop(0, my_nch, zchunk, 0)
        plsc.subcore_barrier()

        def idx_issue(i, sidx, didx, pvv, semi):
            off = base + i * cs
            pltpu.async_copy(src_hbm.at[pl.ds(off, cs)], sidx, semi)
            pltpu.async_copy(dst_hbm.at[pl.ds(off, cs)], didx, semi)
            pltpu.async_copy(p_hbm.at[pl.ds(off, cs)], pvv, semi)

        def idx_wait(i, sidx, didx, pvv, semi):
            off = base + i * cs
            pltpu.make_async_copy(src_hbm.at[pl.ds(off, cs)], sidx, semi).wait()
            pltpu.make_async_copy(dst_hbm.at[pl.ds(off, cs)], didx, semi).wait()
            pltpu.make_async_copy(p_hbm.at[pl.ds(off, cs)], pvv, semi).wait()

        def compute(xvv, pvv, msg, didx, dsv):
            @plsc.parallel_loop(0, cs, unroll=8)
            def row(r):
                prow = pvv[r, :]
                for h in range(d // 16):
                    sl = pl.ds(h * 16, 16)
                    pe = prow.at[jnp.full((16,), h, jnp.int32)].get(
                        mode="promise_in_bounds")
                    msg[r, sl] = xvv[r, sl] * pe
                msg[r, pl.ds(d, 16)] = prow
            offs = list(range(0, cs - 15, 16))
            if cs % 16:
                offs.append(cs - 16)
            for q in offs:
                sl = pl.ds(q, 16)
                dsv[sl] = didx[sl]

        def scat_issue(msg, dsv, sems):
            pltpu.async_copy(msg, acc.at[dsv], sems, add=True)

        def scat_wait(msg, dsv, sems):
            pltpu.make_async_copy(msg, acc.at[dsv], sems).wait()

        idx_issue(0, sidx0, didx0, pvv0, semi0)
        idx_issue(1, sidx1, didx1, pvv1, semi1)
        idx_wait(0, sidx0, didx0, pvv0, semi0)
        pltpu.async_copy(xv_hbm.at[sidx0], xvv0, semg0)

        def body(j, _):
            a = 2 * j
            b = a + 1
            idx_wait(b, sidx1, didx1, pvv1, semi1)
            pltpu.async_copy(xv_hbm.at[sidx1], xvv1, semg1)
            pltpu.make_async_copy(xv_hbm.at[sidx0], xvv0, semg0).wait()

            @pl.when(j > 0)
            def _():
                scat_wait(msg0, ds0, sems0)

            compute(xvv0, pvv0, msg0, didx0, ds0)
            scat_issue(msg0, ds0, sems0)

            @pl.when(a + 2 < ch)
            def _():
                idx_issue(a + 2, sidx0, didx0, pvv0, semi0)

            pltpu.make_async_copy(xv_hbm.at[sidx1], xvv1, semg1).wait()

            @pl.when(j > 0)
            def _():
                scat_wait(msg1, ds1, sems1)

            compute(xvv1, pvv1, msg1, didx1, ds1)
            scat_issue(msg1, ds1, sems1)

            @pl.when(b + 2 < ch)
            def _():
                idx_issue(b + 2, sidx1, didx1, pvv1, semi1)

            @pl.when(a + 2 < ch)
            def _():
                idx_wait(a + 2, sidx0, didx0, pvv0, semi0)
                pltpu.async_copy(xv_hbm.at[sidx0], xvv0, semg0)

            return 0

        lax.fori_loop(0, ch // 2, body, 0)
        scat_wait(msg0, ds0, sems0)
        scat_wait(msg1, ds1, sems1)
        plsc.subcore_barrier()

        def wchunk(j, _):
            row0 = (sid + j * NS) * cs
            pltpu.sync_copy(acc.at[pl.ds(row0, cs)],
                            out_hbm.at[cid, pl.ds(row0, cs)])
            return 0

        lax.fori_loop(0, my_nch, wchunk, 0)

    return k(xv, src, dst, p16)


def kernel(x, edge_index, edge_attr, wq_w, wq_b, wk_w, wv_w, web_w, web_b,
           wew_w, wew_b, wo_w, weo_w, Aw, ln_ng, ln_nb, ln_eg, ln_eb):
    n, d = x.shape
    hd, h, _ = Aw.shape
    src = edge_index[0]
    dst = edge_index[1]

    m16 = jnp.zeros((d, 16), F32).at[
        jnp.arange(d), jnp.arange(d) // hd].set(Aw[:, :, 0].T.reshape(-1))

    wc1 = jnp.concatenate([wew_w, web_w], axis=1).astype(jnp.bfloat16)
    bc1 = jnp.concatenate([wew_b, web_b]).reshape(1, 2 * d)
    wc2 = jnp.concatenate([weo_w, m16, jnp.zeros((d, d - 16), F32)],
                          axis=1).astype(jnp.bfloat16)

    xq, xk, xv = _tc_qkv(x, wq_w, wq_b, wk_w, wv_w)
    g1 = _sc_gather_add(xk, xq, src, dst)
    e_out, p16 = _tc_edge(edge_attr, g1, wc1, bc1, wc2, ln_eg, ln_eb)
    parts = _sc_segment(xv, src, dst, p16)
    u144 = parts[0] + parts[1]
    u = u144[:, :d]
    den128 = jnp.repeat(u144[:, d:d + h], hd, axis=1)
    hh = _tc_node(u, den128, x, wo_w, ln_ng, ln_nb)
    return (hh, e_out)


# speedup vs baseline: 1.0683x; 1.0570x over previous
import functools

import jax
import jax.numpy as jnp
from jax import lax
from jax.experimental import pallas as pl
from jax.experimental.pallas import tpu as pltpu
from jax.experimental.pallas import tpu_sc as plsc

F32 = jnp.float32
NC = 2
NS = 16
NW = NC * NS
C = 80


def _tc_qkv(x, wq_w, wq_b, wk_w, wv_w):
    n, d = x.shape
    bn = 2000

    def body(x_ref, wq_ref, wqb_ref, wk_ref, wv_ref, xq_ref, xk_ref, xv_ref):
        xb = x_ref[...]
        xq_ref[...] = jnp.dot(xb, wq_ref[...], preferred_element_type=F32) + wqb_ref[...]
        xk_ref[...] = jnp.dot(xb, wk_ref[...], preferred_element_type=F32)
        xv_ref[...] = jnp.dot(xb, wv_ref[...], preferred_element_type=F32)

    out = jax.ShapeDtypeStruct((n, d), F32)
    w_spec = pl.BlockSpec((d, d), lambda i: (0, 0))
    b_spec = pl.BlockSpec((1, d), lambda i: (0, 0))
    r_spec = pl.BlockSpec((bn, d), lambda i: (i, 0))
    return pl.pallas_call(
        body,
        grid=(n // bn,),
        in_specs=[r_spec, w_spec, b_spec, w_spec, w_spec],
        out_specs=[r_spec, r_spec, r_spec],
        out_shape=[out, out, out],
    )(x, wq_w, wq_b.reshape(1, d), wk_w, wv_w)


def _edge_s(eab, g1b, wc1_ref, bc1_ref):
    d = eab.shape[1]
    big1 = jnp.dot(eab.astype(jnp.bfloat16), wc1_ref[...],
                   preferred_element_type=F32) + bc1_ref[...]
    t = g1b * big1[:, :d]
    s = jnp.sqrt(jnp.maximum(t, 0.0)) - jnp.sqrt(jnp.maximum(-t, 0.0))
    return jnp.maximum(s + big1[:, d:], 0.0)


def _tc_edge(ea, g1, wc1, bc1, wc2, ln_eg, ln_eb):
    e, d = ea.shape
    be = 4000

    def body(ea_ref, g1_ref, wc1_ref, bc1_ref, wc2_ref, g_ref, b_ref,
             eout_ref, p_ref):
        eab = ea_ref[...]
        s = _edge_s(eab, g1_ref[...], wc1_ref, bc1_ref)
        big2 = jnp.dot(s.astype(jnp.bfloat16), wc2_ref[...],
                       preferred_element_type=F32)
        p_ref[...] = jnp.exp(jnp.clip(big2[:, d:d + 16], -5.0, 5.0))
        eo = big2[:, :d] + eab
        mu = jnp.mean(eo, axis=-1, keepdims=True)
        var = jnp.mean((eo - mu) ** 2, axis=-1, keepdims=True)
        eout_ref[...] = (eo - mu) / jnp.sqrt(var + 1e-5) * g_ref[...] + b_ref[...]

    b_spec = pl.BlockSpec((1, d), lambda i: (0, 0))
    r_spec = pl.BlockSpec((be, d), lambda i: (i, 0))
    return pl.pallas_call(
        body,
        grid=(e // be,),
        in_specs=[r_spec, r_spec,
                  pl.BlockSpec((d, 2 * d), lambda i: (0, 0)),
                  pl.BlockSpec((1, 2 * d), lambda i: (0, 0)),
                  pl.BlockSpec((d, 2 * d), lambda i: (0, 0)),
                  b_spec, b_spec],
        out_specs=[r_spec, pl.BlockSpec((be, 16), lambda i: (i, 0))],
        out_shape=[jax.ShapeDtypeStruct((e, d), F32),
                   jax.ShapeDtypeStruct((e, 16), F32)],
    )(ea, g1, wc1, bc1, wc2, ln_eg.reshape(1, d), ln_eb.reshape(1, d))


def _tc_node(parts, den128, x, wo_w, ln_ng, ln_nb):
    n, d = x.shape

    def body(parts_ref, den_ref, x_ref, wo_ref, g_ref, b_ref, h_ref):
        u = parts_ref[0, :, :d] + parts_ref[1, :, :d]
        xo = u / (den_ref[...] + 1e-16)
        hh = jnp.dot(xo, wo_ref[...], preferred_element_type=F32) + x_ref[...]
        mu = jnp.mean(hh, axis=-1, keepdims=True)
        var = jnp.mean((hh - mu) ** 2, axis=-1, keepdims=True)
        h_ref[...] = (hh - mu) / jnp.sqrt(var + 1e-5) * g_ref[...] + b_ref[...]

    bn = 2000
    dm = parts.shape[2]
    w_spec = pl.BlockSpec((d, d), lambda i: (0, 0))
    b_spec = pl.BlockSpec((1, d), lambda i: (0, 0))
    r_spec = pl.BlockSpec((bn, d), lambda i: (i, 0))
    return pl.pallas_call(
        body,
        grid=(n // bn,),
        in_specs=[pl.BlockSpec((2, bn, dm), lambda i: (0, i, 0)),
                  r_spec, r_spec, w_spec, b_spec, b_spec],
        out_specs=r_spec,
        out_shape=jax.ShapeDtypeStruct((n, d), F32),
    )(parts, den128, x, wo_w, ln_ng.reshape(1, d), ln_nb.reshape(1, d))


def _sc_gather_add(xk, xq, src, dst):
    n, d = xk.shape
    e = src.shape[0]
    ew_ = e // NW
    ch = ew_ // C
    npair = ch // 2
    mesh = plsc.VectorSubcoreMesh(core_axis_name="c", subcore_axis_name="s")

    buf = lambda: [pltpu.VMEM((C,), jnp.int32), pltpu.VMEM((C,), jnp.int32),
                   pltpu.VMEM((C, d), F32), pltpu.VMEM((C, d), F32),
                   pltpu.VMEM((C, d), F32)]

    @functools.partial(
        pl.kernel,
        out_type=jax.ShapeDtypeStruct((e, d), F32),
        mesh=mesh,
        scratch_types=buf() + buf() + [pltpu.SemaphoreType.DMA] * 6,
        compiler_params=pltpu.CompilerParams(use_tc_tiling_on_sc=False,
                                             needs_layout_passes=False),
    )
    def k(xk_hbm, xq_hbm, src_hbm, dst_hbm, g1_hbm,
          sidx0, didx0, ka0, qa0, ob0, sidx1, didx1, ka1, qa1, ob1,
          semi0, semi1, semg0, semg1, semw0, semw1):
        wid = lax.axis_index("c") * NS + lax.axis_index("s")
        base = wid * ew_

        def idx_issue(i, sidx, didx, semi):
            off = base + i * C
            pltpu.async_copy(src_hbm.at[pl.ds(off, C)], sidx, semi)
            pltpu.async_copy(dst_hbm.at[pl.ds(off, C)], didx, semi)

        def idx_wait(i, sidx, didx, semi):
            off = base + i * C
            pltpu.make_async_copy(src_hbm.at[pl.ds(off, C)], sidx, semi).wait()
            pltpu.make_async_copy(dst_hbm.at[pl.ds(off, C)], didx, semi).wait()

        def gat_issue(sidx, didx, ka, qa, semg):
            pltpu.async_copy(xk_hbm.at[sidx], ka, semg)
            pltpu.async_copy(xq_hbm.at[didx], qa, semg)

        def gat_wait(sidx, didx, ka, qa, semg):
            pltpu.make_async_copy(xk_hbm.at[sidx], ka, semg).wait()
            pltpu.make_async_copy(xq_hbm.at[didx], qa, semg).wait()

        def add(ka, qa, ob):
            @plsc.parallel_loop(0, C, unroll=8)
            def row(r):
                for j in range(d // 16):
                    sl = pl.ds(j * 16, 16)
                    ob[r, sl] = ka[r, sl] + qa[r, sl]

        def w_issue(i, ob, semw):
            pltpu.async_copy(ob, g1_hbm.at[pl.ds(base + i * C, C)], semw)

        def w_wait(i, ob, semw):
            pltpu.make_async_copy(ob, g1_hbm.at[pl.ds(base + i * C, C)],
                                  semw).wait()

        idx_issue(0, sidx0, didx0, semi0)
        idx_issue(1, sidx1, didx1, semi1)
        idx_wait(0, sidx0, didx0, semi0)
        gat_issue(sidx0, didx0, ka0, qa0, semg0)

        def body(j, _):
            a = 2 * j
            b = a + 1
            idx_wait(b, sidx1, didx1, semi1)
            gat_issue(sidx1, didx1, ka1, qa1, semg1)
            gat_wait(sidx0, didx0, ka0, qa0, semg0)

            @pl.when(j > 0)
            def _():
                w_wait(a - 2, ob0, semw0)

            add(ka0, qa0, ob0)
            w_issue(a, ob0, semw0)
            idx_issue(a + 2, sidx0, didx0, semi0)
            gat_wait(sidx1, didx1, ka1, qa1, semg1)

            @pl.when(j > 0)
            def _():
                w_wait(b - 2, ob1, semw1)

            add(ka1, qa1, ob1)
            w_issue(b, ob1, semw1)

            @pl.when(b + 2 < ch)
            def _():
                idx_issue(b + 2, sidx1, didx1, semi1)

            idx_wait(a + 2, sidx0, didx0, semi0)
            gat_issue(sidx0, didx0, ka0, qa0, semg0)
            return 0

        lax.fori_loop(0, npair, body, 0)
        gat_wait(sidx0, didx0, ka0, qa0, semg0)
        w_wait(ch - 3, ob0, semw0)
        add(ka0, qa0, ob0)
        w_issue(ch - 1, ob0, semw0)
        w_wait(ch - 1, ob0, semw0)
        w_wait(ch - 2, ob1, semw1)

    return k(xk, xq, src, dst)


def _sc_segment(xv, src, dst, p16):
    n, d = xv.shape
    e = src.shape[0]
    dm = d + 16
    cs = 40
    ew_ = e // NW
    ch = ew_ // cs
    nch = n // cs
    mesh = plsc.VectorSubcoreMesh(core_axis_name="c", subcore_axis_name="s")

    @functools.partial(
        pl.kernel,
        out_type=jax.ShapeDtypeStruct((NC, n, dm), F32),
        mesh=mesh,
        scratch_types=(
            [pltpu.VMEM((cs,), jnp.int32), pltpu.VMEM((cs,), jnp.int32),
             pltpu.VMEM((cs,), jnp.int32), pltpu.VMEM((cs, 16), F32),
             pltpu.VMEM((cs, d), F32), pltpu.VMEM((cs, dm), F32)] * 2
            + [pltpu.VMEM_SHARED((n, dm), F32)]
            + [pltpu.SemaphoreType.DMA] * 6),
        compiler_params=pltpu.CompilerParams(use_tc_tiling_on_sc=False,
                                             needs_layout_passes=False),
    )
    def k(xv_hbm, src_hbm, dst_hbm, p_hbm, out_hbm,
          sidx0, didx0, ds0, pvv0, xvv0, msg0,
          sidx1, didx1, ds1, pvv1, xvv1, msg1,
          acc, semi0, semi1, semg0, semg1, sems0, sems1):
        cid = lax.axis_index("c")
        sid = lax.axis_index("s")
        wid = cid * NS + sid
        base = wid * ew_
        my_nch = (nch - sid + NS - 1) // NS

        def zrow(r, _):
            for j in range(dm // 16):
                msg0[r, pl.ds(j * 16, 16)] = jnp.zeros((16,), F32)
            return 0

        lax.fori_loop(0, cs, zrow, 0)

        def zchunk(j, _):
            pltpu.sync_copy(msg0, acc.at[pl.ds((sid + j * NS) * cs, cs)])
            return 0

        lax.fori_loop(0, my_nch, zchunk, 0)
        plsc.subcore_barrier()

        def idx_issue(i, sidx, didx, pvv, semi):
            off = base + i * cs
            pltpu.async_copy(src_hbm.at[pl.ds(off, cs)], sidx, semi)
            pltpu.async_copy(dst_hbm.at[pl.ds(off, cs)], didx, semi)
            pltpu.async_copy(p_hbm.at[pl.ds(off, cs)], pvv, semi)

        def idx_wait(i, sidx, didx, pvv, semi):
            off = base + i * cs
            pltpu.make_async_copy(src_hbm.at[pl.ds(off, cs)], sidx, semi).wait()
            pltpu.make_async_copy(dst_hbm.at[pl.ds(off, cs)], didx, semi).wait()
            pltpu.make_async_copy(p_hbm.at[pl.ds(off, cs)], pvv, semi).wait()

        def compute(xvv, pvv, msg, didx, dsv):
            @plsc.parallel_loop(0, cs, unroll=8)
            def row(r):
                prow = pvv[r, :]
                for h in range(d // 16):
                    sl = pl.ds(h * 16, 16)
                    pe = prow.at[jnp.full((16,), h, jnp.int32)].get(
                        mode="promise_in_bounds")
                    msg[r, sl] = xvv[r, sl] * pe
                msg[r, pl.ds(d, 16)] = prow
            offs = list(range(0, cs - 15, 16))
            if cs % 16:
                offs.append(cs - 16)
            for q in offs:
                sl = pl.ds(q, 16)
                dsv[sl] = didx[sl]

        def scat_issue(msg, dsv, sems):
            pltpu.async_copy(msg, acc.at[dsv], sems, add=True)

        def scat_wait(msg, dsv, sems):
            pltpu.make_async_copy(msg, acc.at[dsv], sems).wait()

        idx_issue(0, sidx0, didx0, pvv0, semi0)
        idx_issue(1, sidx1, didx1, pvv1, semi1)
        idx_wait(0, sidx0, didx0, pvv0, semi0)
        pltpu.async_copy(xv_hbm.at[sidx0], xvv0, semg0)

        def body(j, _):
            a = 2 * j
            b = a + 1
            idx_wait(b, sidx1, didx1, pvv1, semi1)
            pltpu.async_copy(xv_hbm.at[sidx1], xvv1, semg1)
            pltpu.make_async_copy(xv_hbm.at[sidx0], xvv0, semg0).wait()

            @pl.when(j > 0)
            def _():
                scat_wait(msg0, ds0, sems0)

            compute(xvv0, pvv0, msg0, didx0, ds0)
            scat_issue(msg0, ds0, sems0)

            @pl.when(a + 2 < ch)
            def _():
                idx_issue(a + 2, sidx0, didx0, pvv0, semi0)

            pltpu.make_async_copy(xv_hbm.at[sidx1], xvv1, semg1).wait()

            @pl.when(j > 0)
            def _():
                scat_wait(msg1, ds1, sems1)

            compute(xvv1, pvv1, msg1, didx1, ds1)
            scat_issue(msg1, ds1, sems1)

            @pl.when(b + 2 < ch)
            def _():
                idx_issue(b + 2, sidx1, didx1, pvv1, semi1)

            @pl.when(a + 2 < ch)
            def _():
                idx_wait(a + 2, sidx0, didx0, pvv0, semi0)
                pltpu.async_copy(xv_hbm.at[sidx0], xvv0, semg0)

            return 0

        lax.fori_loop(0, ch // 2, body, 0)
        scat_wait(msg0, ds0, sems0)
        scat_wait(msg1, ds1, sems1)
        plsc.subcore_barrier()

        def wchunk(j, _):
            row0 = (sid + j * NS) * cs
            pltpu.sync_copy(acc.at[pl.ds(row0, cs)],
                            out_hbm.at[cid, pl.ds(row0, cs)])
            return 0

        lax.fori_loop(0, my_nch, wchunk, 0)

    return k(xv, src, dst, p16)


def kernel(x, edge_index, edge_attr, wq_w, wq_b, wk_w, wv_w, web_w, web_b,
           wew_w, wew_b, wo_w, weo_w, Aw, ln_ng, ln_nb, ln_eg, ln_eb):
    n, d = x.shape
    hd, h, _ = Aw.shape
    src = edge_index[0]
    dst = edge_index[1]

    m16 = jnp.zeros((d, 16), F32).at[
        jnp.arange(d), jnp.arange(d) // hd].set(Aw[:, :, 0].T.reshape(-1))

    wc1 = jnp.concatenate([wew_w, web_w], axis=1).astype(jnp.bfloat16)
    bc1 = jnp.concatenate([wew_b, web_b]).reshape(1, 2 * d)
    wc2 = jnp.concatenate([weo_w, m16, jnp.zeros((d, d - 16), F32)],
                          axis=1).astype(jnp.bfloat16)

    xq, xk, xv = _tc_qkv(x, wq_w, wq_b, wk_w, wv_w)
    g1 = _sc_gather_add(xk, xq, src, dst)
    e_out, p16 = _tc_edge(edge_attr, g1, wc1, bc1, wc2, ln_eg, ln_eb)
    parts = _sc_segment(xv, src, dst, p16)
    den128 = jnp.repeat(parts[0, :, d:d + h] + parts[1, :, d:d + h],
                        hd, axis=1)
    hh = _tc_node(parts, den128, x, wo_w, ln_ng, ln_nb)
    return (hh, e_out)


# speedup vs baseline: 1.0973x; 1.0271x over previous
import functools

import jax
import jax.numpy as jnp
from jax import lax
from jax.experimental import pallas as pl
from jax.experimental.pallas import tpu as pltpu
from jax.experimental.pallas import tpu_sc as plsc

F32 = jnp.float32
NC = 2
NS = 16
NW = NC * NS
C = 80


def _tc_qkv(x, wq_w, wq_b, wk_w, wv_w):
    n, d = x.shape
    bn = 2000

    def body(x_ref, wq_ref, wqb_ref, wk_ref, wv_ref, xq_ref, xk_ref, xv_ref):
        xb = x_ref[...]
        xq_ref[...] = jnp.dot(xb, wq_ref[...], preferred_element_type=F32) + wqb_ref[...]
        xk_ref[...] = jnp.dot(xb, wk_ref[...], preferred_element_type=F32)
        xv_ref[...] = jnp.dot(xb, wv_ref[...], preferred_element_type=F32)

    out = jax.ShapeDtypeStruct((n, d), F32)
    w_spec = pl.BlockSpec((d, d), lambda i: (0, 0))
    b_spec = pl.BlockSpec((1, d), lambda i: (0, 0))
    r_spec = pl.BlockSpec((bn, d), lambda i: (i, 0))
    return pl.pallas_call(
        body,
        grid=(n // bn,),
        in_specs=[r_spec, w_spec, b_spec, w_spec, w_spec],
        out_specs=[r_spec, r_spec, r_spec],
        out_shape=[out, out, out],
    )(x, wq_w, wq_b.reshape(1, d), wk_w, wv_w)


def _edge_s(eab, g1b, wc1_ref, bc1_ref):
    d = eab.shape[1]
    big1 = jnp.dot(eab.astype(jnp.bfloat16), wc1_ref[...],
                   preferred_element_type=F32) + bc1_ref[...]
    t = g1b * big1[:, :d]
    s = jnp.sqrt(jnp.maximum(t, 0.0)) - jnp.sqrt(jnp.maximum(-t, 0.0))
    return jnp.maximum(s + big1[:, d:], 0.0)


def _tc_edge(ea, g1, wc1, bc1, wc2, ln_eg, ln_eb):
    e, d = ea.shape
    be = 8000

    def body(ea_ref, g1_ref, wc1_ref, bc1_ref, wc2_ref, g_ref, b_ref,
             eout_ref, p_ref):
        eab = ea_ref[...]
        s = _edge_s(eab, g1_ref[...], wc1_ref, bc1_ref)
        big2 = jnp.dot(s.astype(jnp.bfloat16), wc2_ref[...],
                       preferred_element_type=F32)
        p_ref[...] = jnp.exp(jnp.clip(big2[:, d:d + 16], -5.0, 5.0))
        eo = big2[:, :d] + eab
        mu = jnp.mean(eo, axis=-1, keepdims=True)
        var = jnp.mean((eo - mu) ** 2, axis=-1, keepdims=True)
        eout_ref[...] = (eo - mu) / jnp.sqrt(var + 1e-5) * g_ref[...] + b_ref[...]

    b_spec = pl.BlockSpec((1, d), lambda i: (0, 0))
    r_spec = pl.BlockSpec((be, d), lambda i: (i, 0))
    return pl.pallas_call(
        body,
        grid=(e // be,),
        in_specs=[r_spec, r_spec,
                  pl.BlockSpec((d, 2 * d), lambda i: (0, 0)),
                  pl.BlockSpec((1, 2 * d), lambda i: (0, 0)),
                  pl.BlockSpec((d, 2 * d), lambda i: (0, 0)),
                  b_spec, b_spec],
        out_specs=[r_spec, pl.BlockSpec((be, 16), lambda i: (i, 0))],
        out_shape=[jax.ShapeDtypeStruct((e, d), F32),
                   jax.ShapeDtypeStruct((e, 16), F32)],
    )(ea, g1, wc1, bc1, wc2, ln_eg.reshape(1, d), ln_eb.reshape(1, d))


def _tc_node(parts, den128, x, wo_w, ln_ng, ln_nb):
    n, d = x.shape

    def body(parts_ref, den_ref, x_ref, wo_ref, g_ref, b_ref, h_ref):
        u = parts_ref[0, :, :d] + parts_ref[1, :, :d]
        xo = u / (den_ref[...] + 1e-16)
        hh = jnp.dot(xo, wo_ref[...], preferred_element_type=F32) + x_ref[...]
        mu = jnp.mean(hh, axis=-1, keepdims=True)
        var = jnp.mean((hh - mu) ** 2, axis=-1, keepdims=True)
        h_ref[...] = (hh - mu) / jnp.sqrt(var + 1e-5) * g_ref[...] + b_ref[...]

    bn = 2000
    dm = parts.shape[2]
    w_spec = pl.BlockSpec((d, d), lambda i: (0, 0))
    b_spec = pl.BlockSpec((1, d), lambda i: (0, 0))
    r_spec = pl.BlockSpec((bn, d), lambda i: (i, 0))
    return pl.pallas_call(
        body,
        grid=(n // bn,),
        in_specs=[pl.BlockSpec((2, bn, dm), lambda i: (0, i, 0)),
                  r_spec, r_spec, w_spec, b_spec, b_spec],
        out_specs=r_spec,
        out_shape=jax.ShapeDtypeStruct((n, d), F32),
    )(parts, den128, x, wo_w, ln_ng.reshape(1, d), ln_nb.reshape(1, d))


def _sc_gather_add(xk, xq, src, dst):
    n, d = xk.shape
    e = src.shape[0]
    ew_ = e // NW
    ch = ew_ // C
    npair = ch // 2
    mesh = plsc.VectorSubcoreMesh(core_axis_name="c", subcore_axis_name="s")

    buf = lambda: [pltpu.VMEM((C,), jnp.int32), pltpu.VMEM((C,), jnp.int32),
                   pltpu.VMEM((C, d), F32), pltpu.VMEM((C, d), F32),
                   pltpu.VMEM((C, d), F32)]

    @functools.partial(
        pl.kernel,
        out_type=jax.ShapeDtypeStruct((e, d), F32),
        mesh=mesh,
        scratch_types=buf() + buf() + [pltpu.SemaphoreType.DMA] * 6,
        compiler_params=pltpu.CompilerParams(use_tc_tiling_on_sc=False,
                                             needs_layout_passes=False),
    )
    def k(xk_hbm, xq_hbm, src_hbm, dst_hbm, g1_hbm,
          sidx0, didx0, ka0, qa0, ob0, sidx1, didx1, ka1, qa1, ob1,
          semi0, semi1, semg0, semg1, semw0, semw1):
        wid = lax.axis_index("c") * NS + lax.axis_index("s")
        base = wid * ew_

        def idx_issue(i, sidx, didx, semi):
            off = base + i * C
            pltpu.async_copy(src_hbm.at[pl.ds(off, C)], sidx, semi)
            pltpu.async_copy(dst_hbm.at[pl.ds(off, C)], didx, semi)

        def idx_wait(i, sidx, didx, semi):
            off = base + i * C
            pltpu.make_async_copy(src_hbm.at[pl.ds(off, C)], sidx, semi).wait()
            pltpu.make_async_copy(dst_hbm.at[pl.ds(off, C)], didx, semi).wait()

        def gat_issue(sidx, didx, ka, qa, semg):
            pltpu.async_copy(xk_hbm.at[sidx], ka, semg)
            pltpu.async_copy(xq_hbm.at[didx], qa, semg)

        def gat_wait(sidx, didx, ka, qa, semg):
            pltpu.make_async_copy(xk_hbm.at[sidx], ka, semg).wait()
            pltpu.make_async_copy(xq_hbm.at[didx], qa, semg).wait()

        def add(ka, qa, ob):
            @plsc.parallel_loop(0, C, unroll=8)
            def row(r):
                for j in range(d // 16):
                    sl = pl.ds(j * 16, 16)
                    ob[r, sl] = ka[r, sl] + qa[r, sl]

        def w_issue(i, ob, semw):
            pltpu.async_copy(ob, g1_hbm.at[pl.ds(base + i * C, C)], semw)

        def w_wait(i, ob, semw):
            pltpu.make_async_copy(ob, g1_hbm.at[pl.ds(base + i * C, C)],
                                  semw).wait()

        idx_issue(0, sidx0, didx0, semi0)
        idx_issue(1, sidx1, didx1, semi1)
        idx_wait(0, sidx0, didx0, semi0)
        gat_issue(sidx0, didx0, ka0, qa0, semg0)

        def body(j, _):
            a = 2 * j
            b = a + 1
            idx_wait(b, sidx1, didx1, semi1)
            gat_issue(sidx1, didx1, ka1, qa1, semg1)
            gat_wait(sidx0, didx0, ka0, qa0, semg0)

            @pl.when(j > 0)
            def _():
                w_wait(a - 2, ob0, semw0)

            add(ka0, qa0, ob0)
            w_issue(a, ob0, semw0)
            idx_issue(a + 2, sidx0, didx0, semi0)
            gat_wait(sidx1, didx1, ka1, qa1, semg1)

            @pl.when(j > 0)
            def _():
                w_wait(b - 2, ob1, semw1)

            add(ka1, qa1, ob1)
            w_issue(b, ob1, semw1)

            @pl.when(b + 2 < ch)
            def _():
                idx_issue(b + 2, sidx1, didx1, semi1)

            idx_wait(a + 2, sidx0, didx0, semi0)
            gat_issue(sidx0, didx0, ka0, qa0, semg0)
            return 0

        lax.fori_loop(0, npair, body, 0)
        gat_wait(sidx0, didx0, ka0, qa0, semg0)
        w_wait(ch - 3, ob0, semw0)
        add(ka0, qa0, ob0)
        w_issue(ch - 1, ob0, semw0)
        w_wait(ch - 1, ob0, semw0)
        w_wait(ch - 2, ob1, semw1)

    return k(xk, xq, src, dst)


def _sc_segment(xv, src, dst, p16):
    n, d = xv.shape
    e = src.shape[0]
    dm = d + 16
    cs = 40
    ew_ = e // NW
    ch = ew_ // cs
    nch = n // cs
    mesh = plsc.VectorSubcoreMesh(core_axis_name="c", subcore_axis_name="s")

    @functools.partial(
        pl.kernel,
        out_type=jax.ShapeDtypeStruct((NC, n, dm), F32),
        mesh=mesh,
        scratch_types=(
            [pltpu.VMEM((cs,), jnp.int32), pltpu.VMEM((cs,), jnp.int32),
             pltpu.VMEM((cs,), jnp.int32), pltpu.VMEM((cs, 16), F32),
             pltpu.VMEM((cs, d), F32), pltpu.VMEM((cs, dm), F32)] * 2
            + [pltpu.VMEM_SHARED((n, dm), F32)]
            + [pltpu.SemaphoreType.DMA] * 6),
        compiler_params=pltpu.CompilerParams(use_tc_tiling_on_sc=False,
                                             needs_layout_passes=False),
    )
    def k(xv_hbm, src_hbm, dst_hbm, p_hbm, out_hbm,
          sidx0, didx0, ds0, pvv0, xvv0, msg0,
          sidx1, didx1, ds1, pvv1, xvv1, msg1,
          acc, semi0, semi1, semg0, semg1, sems0, sems1):
        cid = lax.axis_index("c")
        sid = lax.axis_index("s")
        wid = cid * NS + sid
        base = wid * ew_
        my_nch = (nch - sid + NS - 1) // NS

        def zrow(r, _):
            for j in range(dm // 16):
                msg0[r, pl.ds(j * 16, 16)] = jnp.zeros((16,), F32)
            return 0

        lax.fori_loop(0, cs, zrow, 0)

        def zchunk(j, _):
            pltpu.sync_copy(msg0, acc.at[pl.ds((sid + j * NS) * cs, cs)])
            return 0

        lax.fori_loop(0, my_nch, zchunk, 0)
        plsc.subcore_barrier()

        def idx_issue(i, sidx, didx, pvv, semi):
            off = base + i * cs
            pltpu.async_copy(src_hbm.at[pl.ds(off, cs)], sidx, semi)
            pltpu.async_copy(dst_hbm.at[pl.ds(off, cs)], didx, semi)
            pltpu.async_copy(p_hbm.at[pl.ds(off, cs)], pvv, semi)

        def idx_wait(i, sidx, didx, pvv, semi):
            off = base + i * cs
            pltpu.make_async_copy(src_hbm.at[pl.ds(off, cs)], sidx, semi).wait()
            pltpu.make_async_copy(dst_hbm.at[pl.ds(off, cs)], didx, semi).wait()
            pltpu.make_async_copy(p_hbm.at[pl.ds(off, cs)], pvv, semi).wait()

        def compute(xvv, pvv, msg, didx, dsv):
            @plsc.parallel_loop(0, cs, unroll=8)
            def row(r):
                prow = pvv[r, :]
                for h in range(d // 16):
                    sl = pl.ds(h * 16, 16)
                    pe = prow.at[jnp.full((16,), h, jnp.int32)].get(
                        mode="promise_in_bounds")
                    msg[r, sl] = xvv[r, sl] * pe
                msg[r, pl.ds(d, 16)] = prow
            offs = list(range(0, cs - 15, 16))
            if cs % 16:
                offs.append(cs - 16)
            for q in offs:
                sl = pl.ds(q, 16)
                dsv[sl] = didx[sl]

        def scat_issue(msg, dsv, sems):
            pltpu.async_copy(msg, acc.at[dsv], sems, add=True)

        def scat_wait(msg, dsv, sems):
            pltpu.make_async_copy(msg, acc.at[dsv], sems).wait()

        idx_issue(0, sidx0, didx0, pvv0, semi0)
        idx_issue(1, sidx1, didx1, pvv1, semi1)
        idx_wait(0, sidx0, didx0, pvv0, semi0)
        pltpu.async_copy(xv_hbm.at[sidx0], xvv0, semg0)

        def body(j, _):
            a = 2 * j
            b = a + 1
            idx_wait(b, sidx1, didx1, pvv1, semi1)
            pltpu.async_copy(xv_hbm.at[sidx1], xvv1, semg1)
            pltpu.make_async_copy(xv_hbm.at[sidx0], xvv0, semg0).wait()

            @pl.when(j > 0)
            def _():
                scat_wait(msg0, ds0, sems0)

            compute(xvv0, pvv0, msg0, didx0, ds0)
            scat_issue(msg0, ds0, sems0)

            @pl.when(a + 2 < ch)
            def _():
                idx_issue(a + 2, sidx0, didx0, pvv0, semi0)

            pltpu.make_async_copy(xv_hbm.at[sidx1], xvv1, semg1).wait()

            @pl.when(j > 0)
            def _():
                scat_wait(msg1, ds1, sems1)

            compute(xvv1, pvv1, msg1, didx1, ds1)
            scat_issue(msg1, ds1, sems1)

            @pl.when(b + 2 < ch)
            def _():
                idx_issue(b + 2, sidx1, didx1, pvv1, semi1)

            @pl.when(a + 2 < ch)
            def _():
                idx_wait(a + 2, sidx0, didx0, pvv0, semi0)
                pltpu.async_copy(xv_hbm.at[sidx0], xvv0, semg0)

            return 0

        lax.fori_loop(0, ch // 2, body, 0)
        scat_wait(msg0, ds0, sems0)
        scat_wait(msg1, ds1, sems1)
        plsc.subcore_barrier()

        def wchunk(j, _):
            row0 = (sid + j * NS) * cs
            pltpu.sync_copy(acc.at[pl.ds(row0, cs)],
                            out_hbm.at[cid, pl.ds(row0, cs)])
            return 0

        lax.fori_loop(0, my_nch, wchunk, 0)

    return k(xv, src, dst, p16)


def kernel(x, edge_index, edge_attr, wq_w, wq_b, wk_w, wv_w, web_w, web_b,
           wew_w, wew_b, wo_w, weo_w, Aw, ln_ng, ln_nb, ln_eg, ln_eb):
    n, d = x.shape
    hd, h, _ = Aw.shape
    src = edge_index[0]
    dst = edge_index[1]

    m16 = jnp.zeros((d, 16), F32).at[
        jnp.arange(d), jnp.arange(d) // hd].set(Aw[:, :, 0].T.reshape(-1))

    wc1 = jnp.concatenate([wew_w, web_w], axis=1).astype(jnp.bfloat16)
    bc1 = jnp.concatenate([wew_b, web_b]).reshape(1, 2 * d)
    wc2 = jnp.concatenate([weo_w, m16, jnp.zeros((d, d - 16), F32)],
                          axis=1).astype(jnp.bfloat16)

    xq, xk, xv = _tc_qkv(x, wq_w, wq_b, wk_w, wv_w)
    g1 = _sc_gather_add(xk, xq, src, dst)
    e_out, p16 = _tc_edge(edge_attr, g1, wc1, bc1, wc2, ln_eg, ln_eb)
    parts = _sc_segment(xv, src, dst, p16)
    den128 = jnp.repeat(parts[0, :, d:d + h] + parts[1, :, d:d + h],
                        hd, axis=1)
    hh = _tc_node(parts, den128, x, wo_w, ln_ng, ln_nb)
    return (hh, e_out)
